# Initial kernel scaffold; baseline (speedup 1.0000x reference)
import jax
import jax.numpy as jnp
from jax.experimental import pallas as pl


def kernel(x, edge_attr, edge_index, batch, Wng0, bng0, Wng1, bng1, Wng2, bng2, Wi, bi, Wm, bm, alpha, We, be, gamma, beta, Wfe0, bfe0, Wfe1, bfe1, Wfe2, bfe2, Wa, ba, Wl, bl):
    raise NotImplementedError("write your pallas kernel here")


# speedup vs baseline: 12.8003x; 12.8003x over previous
import functools

import jax
import jax.numpy as jnp
from jax import lax
from jax.experimental import pallas as pl
from jax.experimental.pallas import tpu as pltpu
from jax.experimental.pallas import tpu_sc as plsc

N = 10000
E = 320000
G = 100
GP = 128
DIN = 128
DE = 16
H = 64
NL = 4
EPS = 1e-5

NC = 2
NS = 16
NW = NC * NS
EPW = E // NW
CH = 125
NCHUNK = EPW // CH
NB = 4
NP = 10240
RPT = NP // NS
ZCH = 128

f32 = jnp.float32


def _sc_gather_body(h_hbm, src_hbm, dst_hbm, out_hbm,
                    src_v, dst_v, r0, r1, r2, r3, zbuf, acc, s0, s1, s2, s3):
    cid = lax.axis_index("c")
    sid = lax.axis_index("s")
    wid = sid * NC + cid
    rows = [r0, r1, r2, r3]
    sems = [s0, s1, s2, s3]

    def _zrow(r, _):
        for j in range(H // 16):
            zbuf[r, pl.ds(j * 16, 16)] = jnp.zeros((16,), f32)
        return _
    lax.fori_loop(0, ZCH, _zrow, None)
    row0 = sid * RPT
    for k in range(RPT // ZCH):
        pltpu.sync_copy(zbuf, acc.at[pl.ds(row0 + k * ZCH, ZCH)])
    plsc.subcore_barrier()

    pltpu.sync_copy(src_hbm.at[wid], src_v)
    pltpu.sync_copy(dst_hbm.at[wid], dst_v)

    for b in range(NB):
        pltpu.async_copy(h_hbm.at[src_v.at[b]], rows[b], sems[b])

    def _grp(g, _):
        base = g * NB
        for b in range(NB):
            c = base + b
            pltpu.make_async_copy(h_hbm.at[src_v.at[c]], rows[b], sems[b]).wait()
            pltpu.sync_copy(rows[b], acc.at[dst_v.at[c]], add=True)

            @pl.when(c + NB < NCHUNK)
            def _():
                pltpu.async_copy(h_hbm.at[src_v.at[c + NB]], rows[b], sems[b])
        return _
    lax.fori_loop(0, NCHUNK // NB, _grp, None)

    plsc.subcore_barrier()
    pltpu.sync_copy(acc.at[pl.ds(row0, RPT)], out_hbm.at[cid, pl.ds(row0, RPT)])


def _sc_gather(h, src3, dst3):
    mesh = plsc.VectorSubcoreMesh(core_axis_name="c", subcore_axis_name="s")
    kfn = pl.kernel(
        _sc_gather_body,
        out_type=jax.ShapeDtypeStruct((NC, NP, H), f32),
        mesh=mesh,
        scratch_types=(
            [pltpu.VMEM((NCHUNK, CH), jnp.int32),
             pltpu.VMEM((NCHUNK, CH), jnp.int32)]
            + [pltpu.VMEM((CH, H), f32) for _ in range(NB)]
            + [pltpu.VMEM((ZCH, H), f32),
               pltpu.VMEM_SHARED((NP, H), f32)]
            + [pltpu.SemaphoreType.DMA for _ in range(NB)]
        ),
        compiler_params=pltpu.CompilerParams(use_tc_tiling_on_sc=False),
    )
    return kfn(h, src3, dst3)


DA = 2 * DE


def _sc_edge_body(ea_hbm, dst_hbm, out_hbm,
                  dst_v, e0, e1, e2, e3, zbuf, acc,
                  s0, s1, s2, s3):
    cid = lax.axis_index("c")
    sid = lax.axis_index("s")
    wid = sid * NC + cid
    ebufs = [e0, e1, e2, e3]
    sems = [s0, s1, s2, s3]

    def _fill(r, _):
        for j in range(DA // 16):
            zbuf[r, pl.ds(j * 16, 16)] = jnp.zeros((16,), f32)
        return _
    lax.fori_loop(0, ZCH, _fill, None)
    row0 = sid * RPT
    for k in range(RPT // ZCH):
        pltpu.sync_copy(zbuf, acc.at[pl.ds(row0 + k * ZCH, ZCH)])
    plsc.subcore_barrier()

    pltpu.sync_copy(dst_hbm.at[wid], dst_v)

    for b in range(NB):
        pltpu.async_copy(ea_hbm.at[wid, b], ebufs[b], sems[b])

    def _grp(g, _):
        base = g * NB
        for b in range(NB):
            c = base + b
            pltpu.make_async_copy(ea_hbm.at[wid, c], ebufs[b], sems[b]).wait()
            pltpu.sync_copy(ebufs[b], acc.at[dst_v.at[c]], add=True)

            @pl.when(c + NB < NCHUNK)
            def _():
                pltpu.async_copy(ea_hbm.at[wid, c + NB], ebufs[b], sems[b])
        return _
    lax.fori_loop(0, NCHUNK // NB, _grp, None)

    plsc.subcore_barrier()
    pltpu.sync_copy(acc.at[pl.ds(row0, RPT)], out_hbm.at[cid, pl.ds(row0, RPT)])


def _sc_edge(edge_attr4, dst3):
    mesh = plsc.VectorSubcoreMesh(core_axis_name="c", subcore_axis_name="s")
    kfn = pl.kernel(
        _sc_edge_body,
        out_type=jax.ShapeDtypeStruct((NC, NP, DA), f32),
        mesh=mesh,
        scratch_types=(
            [pltpu.VMEM((NCHUNK, CH), jnp.int32)]
            + [pltpu.VMEM((CH, DA), f32) for _ in range(NB)]
            + [pltpu.VMEM((ZCH, DA), f32),
               pltpu.VMEM_SHARED((NP, DA), f32)]
            + [pltpu.SemaphoreType.DMA for _ in range(NB)]
        ),
        compiler_params=pltpu.CompilerParams(use_tc_tiling_on_sc=False),
    )
    return kfn(edge_attr4, dst3)


bf16 = jnp.bfloat16


def _mm1(a, b):
    return lax.dot_general(a, b, (((1,), (0,)), ((), ())),
                           preferred_element_type=f32,
                           precision=lax.Precision.DEFAULT)


def _mm(a, b):
    a_hi = a.astype(bf16)
    a_lo = (a - a_hi.astype(f32)).astype(bf16)
    b_hi = b.astype(bf16)
    b_lo = (b - b_hi.astype(f32)).astype(bf16)
    return _mm1(a_hi, b_hi) + _mm1(a_lo, b_hi) + _mm1(a_hi, b_lo)


def _mm_a2(a, b_bf):
    a_hi = a.astype(bf16)
    a_lo = (a - a_hi.astype(f32)).astype(bf16)
    return _mm1(a_hi, b_bf) + _mm1(a_lo, b_bf)


def _pool_mm(mask_b, u):
    u_hi = u.astype(bf16)
    u_lo = (u - u_hi.astype(f32)).astype(bf16)
    return _mm1(mask_b, u_hi) + _mm1(mask_b, u_lo)


def _ext_mlp(gmean, W0, b0, W1, b1, W2, b2):
    o = _mm(gmean, W0[...]) + b0[...]
    hh = jnp.maximum(_mm(o, W1[...]) + b1[...], 0.0)
    return _mm(hh, W2[...]) + b2[...]


def _graph_mean(mtb, cnt, u):
    gsum = _pool_mm(mtb, u)
    return gsum / jnp.maximum(cnt, 1.0)


def _tc_pre_body(x_ref, bcol_ref, brow_ref, sides_ref,
                 Wng0, bng0, Wng1, bng1, Wng2, bng2, Wi, bi,
                 Wm0, bm0, al0, We0, be0,
                 out0_ref, vh_ref, cm_ref, mtb_ref, cnt_ref):
    x = x_ref[...]
    brow = brow_ref[...]
    bcol = bcol_ref[...]
    sides = sides_ref[...]
    ea = sides[:, 0:DE] + sides[:, DE:2 * DE]
    deg1 = sides[:, 2 * DE:2 * DE + 1] + sides[:, 3 * DE:3 * DE + 1]

    gids = lax.broadcasted_iota(jnp.int32, (GP, 1), 0)
    mtb = jnp.where(gids == brow, 1.0, 0.0).astype(bf16)
    mtb_ref[...] = mtb
    cnt = jnp.sum(mtb.astype(f32), axis=1, keepdims=True)
    cnt_ref[...] = cnt

    gmean_x = _graph_mean(mtb, cnt, x)
    out0_ref[...] = _ext_mlp(gmean_x, Wng0, bng0, Wng1, bng1, Wng2, bng2)

    gsides = _pool_mm(mtb, sides)
    gdeg = gsides[:, 2 * DE:2 * DE + 1] + gsides[:, 3 * DE:3 * DE + 1]
    inv_g = 1.0 / jnp.maximum(gdeg / jnp.maximum(cnt, 1.0), 1e-6)

    grow = lax.broadcasted_iota(jnp.int32, (1, GP), 1)
    mnb = jnp.where(bcol == grow, 1.0, 0.0).astype(bf16)
    iv_hi = inv_g.astype(bf16)
    iv_lo = (inv_g - iv_hi.astype(f32)).astype(bf16)
    invb = (_mm1(mnb, jnp.broadcast_to(iv_hi, (GP, DE)))
            + _mm1(mnb, jnp.broadcast_to(iv_lo, (GP, DE))))
    deg16 = jnp.broadcast_to(deg1, (N, DE))

    v = _mm(x, Wi[...]) + bi[...]
    hh = _mm(v, Wm0[...]) + bm0[...]
    cc = (deg1 * (hh * al0[...]) + _mm_a2(ea, We0[...].astype(bf16))
          + deg1 * be0[...])
    meta = jnp.concatenate([ea, deg16, invb, deg16], axis=1)
    vh_ref[...] = jnp.concatenate([v, hh], axis=1)
    cm_ref[...] = jnp.concatenate([cc, meta], axis=1)


def _layer_update(scat, cm, vh):
    s = scat[:, 0:H] + scat[:, H:2 * H]
    cc = cm[:, 0:H]
    inv1 = cm[:, H + 2 * DE:H + 2 * DE + 1]
    v = vh[:, 0:H]
    return (s + cc) * inv1 + v


def _tc_mid_body(scat_ref, cm_ref, vh_ref, mtb_ref, cnt_ref,
                 gam, bet, Wm, bm, al, We, be,
                 Wfe0, bfe0, Wfe1, bfe1, Wfe2, bfe2,
                 ext_ref, vh_out_ref, cm_out_ref):
    cm = cm_ref[...]
    u = _layer_update(scat_ref[...], cm, vh_ref[...])
    gmean = _graph_mean(mtb_ref[...], cnt_ref[...], u)
    ext_ref[...] = _ext_mlp(gmean, Wfe0, bfe0, Wfe1, bfe1, Wfe2, bfe2)

    m = jnp.mean(u, axis=0, keepdims=True)
    var = jnp.mean((u - m) ** 2, axis=0, keepdims=True)
    vn = (u - m) / jnp.sqrt(var + EPS) * gam[...] + bet[...]

    meta = cm[:, H:]
    ea = meta[:, 0:DE]
    deg1 = meta[:, DE:DE + 1]
    hh = _mm(vn, Wm[...]) + bm[...]
    cc = (deg1 * (hh * al[...]) + _mm_a2(ea, We[...].astype(bf16))
          + deg1 * be[...])
    vh_out_ref[...] = jnp.concatenate([vn, hh], axis=1)
    cm_out_ref[...] = jnp.concatenate([cc, meta], axis=1)


def _tc_final_body(scat_ref, cm_ref, vh_ref, mtb_ref, cnt_ref, out0_ref,
                   e0, e1, e2,
                   Wfe0, bfe0, Wfe1, bfe1, Wfe2, bfe2, Wa, ba, Wl, bl,
                   res_ref):
    u = _layer_update(scat_ref[...], cm_ref[...], vh_ref[...])
    gmean = _graph_mean(mtb_ref[...], cnt_ref[...], u)
    e3 = _ext_mlp(gmean, Wfe0, bfe0, Wfe1, bfe1, Wfe2, bfe2)
    out = out0_ref[...] + (e0[...] + e1[...] + e2[...] + e3) * (1.0 / NL)
    o2 = jnp.maximum(_mm(out, Wa[...]) + ba[...], 0.0) + out
    res_ref[...] = _mm(o2, Wl[...]) + bl[...]


def _sds(shape):
    return jax.ShapeDtypeStruct(shape, f32)


def kernel(x, edge_attr, edge_index, batch,
           Wng0, bng0, Wng1, bng1, Wng2, bng2, Wi, bi,
           Wm, bm, alpha, We, be, gamma, beta,
           Wfe0, bfe0, Wfe1, bfe1, Wfe2, bfe2, Wa, ba, Wl, bl):
    src3 = edge_index[0].reshape(NW, NCHUNK, CH)
    dst3 = edge_index[1].reshape(NW, NCHUNK, CH)
    ea_r = edge_attr.astype(jnp.bfloat16).astype(jnp.float32)
    ea4 = jnp.concatenate(
        [ea_r, jnp.ones((E, DE), jnp.float32)], axis=1
    ).reshape(NW, NCHUNK, CH, DA)
    bcol = batch.reshape(N, 1)
    brow = batch.reshape(1, N)

    def r2(b):
        return b.reshape(1, -1)

    eadg = _sc_edge(ea4, dst3)
    sides = jnp.concatenate([eadg[0, :N, 0:DE], eadg[1, :N, 0:DE],
                             eadg[0, :N, DE:DA], eadg[1, :N, DE:DA]], axis=1)

    out0, vh, cm, mtb, cnt = pl.pallas_call(
        _tc_pre_body,
        out_shape=(_sds((GP, H)), _sds((N, 2 * H)), _sds((N, 2 * H)),
                   jax.ShapeDtypeStruct((GP, N), jnp.bfloat16), _sds((GP, 1))),
    )(x, bcol, brow, sides,
      Wng0, r2(bng0), Wng1, r2(bng1), Wng2, r2(bng2), Wi, r2(bi),
      Wm[0], r2(bm[0]), r2(alpha[0]), We[0], r2(be[0]))

    exts = []
    for i in range(NL - 1):
        Sp = _sc_gather(vh[:, H:], src3, dst3)
        scat = jnp.concatenate([Sp[0, :N], Sp[1, :N]], axis=1)
        ext_i, vh, cm = pl.pallas_call(
            _tc_mid_body,
            out_shape=(_sds((GP, H)), _sds((N, 2 * H)), _sds((N, 2 * H))),
        )(scat, cm, vh, mtb, cnt,
          r2(gamma[i]), r2(beta[i]),
          Wm[i + 1], r2(bm[i + 1]), r2(alpha[i + 1]), We[i + 1], r2(be[i + 1]),
          Wfe0, r2(bfe0), Wfe1, r2(bfe1), Wfe2, r2(bfe2))
        exts.append(ext_i)

    Sp = _sc_gather(vh[:, H:], src3, dst3)
    scat = jnp.concatenate([Sp[0, :N], Sp[1, :N]], axis=1)
    res = pl.pallas_call(
        _tc_final_body,
        out_shape=_sds((GP, 1)),
    )(scat, cm, vh, mtb, cnt, out0, exts[0], exts[1], exts[2],
      Wfe0, r2(bfe0), Wfe1, r2(bfe1), Wfe2, r2(bfe2),
      Wa, r2(ba), Wl, r2(bl))

    return res[:G]


# speedup vs baseline: 15.4320x; 1.2056x over previous
import functools

import jax
import jax.numpy as jnp
from jax import lax
from jax.experimental import pallas as pl
from jax.experimental.pallas import tpu as pltpu
from jax.experimental.pallas import tpu_sc as plsc

N = 10000
E = 320000
G = 100
GP = 128
DIN = 128
DE = 16
H = 64
NL = 4
EPS = 1e-5

NC = 2
NS = 16
NW = NC * NS
EPW = E // NW
CH = 125
NCHUNK = EPW // CH
NB = 4
NP = 10240
RPT = NP // NS
ZCH = 128

f32 = jnp.float32


def _sc_gather_body(h_hbm, src_hbm, dst_hbm, out_hbm,
                    src_v, dst_v, r0, r1, r2, r3, zbuf, acc,
                    s0, s1, s2, s3):
    cid = lax.axis_index("c")
    sid = lax.axis_index("s")
    wid = sid * NC + cid
    rows = [r0, r1, r2, r3]
    sems = [s0, s1, s2, s3]

    def _zrow(r, _):
        for j in range(H // 16):
            zbuf[r, pl.ds(j * 16, 16)] = jnp.zeros((16,), f32)
        return _
    lax.fori_loop(0, ZCH, _zrow, None)
    row0 = sid * RPT
    for k in range(RPT // ZCH):
        pltpu.sync_copy(zbuf, acc.at[pl.ds(row0 + k * ZCH, ZCH)])
    plsc.subcore_barrier()

    pltpu.sync_copy(src_hbm.at[wid], src_v)
    pltpu.sync_copy(dst_hbm.at[wid], dst_v)

    for b in range(NB):
        pltpu.async_copy(h_hbm.at[src_v.at[b]], rows[b], sems[b])

    def _grp(g, _):
        base = g * NB
        for b in range(NB):
            c = base + b
            pltpu.make_async_copy(h_hbm.at[src_v.at[c]], rows[b], sems[b]).wait()
            pltpu.sync_copy(rows[b], acc.at[dst_v.at[c]], add=True)

            @pl.when(c + NB < NCHUNK)
            def _():
                pltpu.async_copy(h_hbm.at[src_v.at[c + NB]], rows[b], sems[b])
        return _
    lax.fori_loop(0, NCHUNK // NB, _grp, None)

    plsc.subcore_barrier()
    pltpu.sync_copy(acc.at[pl.ds(row0, RPT)], out_hbm.at[cid, pl.ds(row0, RPT)])


def _sc_gather(h, src3, dst3):
    mesh = plsc.VectorSubcoreMesh(core_axis_name="c", subcore_axis_name="s")
    kfn = pl.kernel(
        _sc_gather_body,
        out_type=jax.ShapeDtypeStruct((NC, NP, H), f32),
        mesh=mesh,
        scratch_types=(
            [pltpu.VMEM((NCHUNK, CH), jnp.int32),
             pltpu.VMEM((NCHUNK, CH), jnp.int32)]
            + [pltpu.VMEM((CH, H), f32) for _ in range(NB)]
            + [pltpu.VMEM((ZCH, H), f32),
               pltpu.VMEM_SHARED((NP, H), f32)]
            + [pltpu.SemaphoreType.DMA for _ in range(NB)]
        ),
        compiler_params=pltpu.CompilerParams(use_tc_tiling_on_sc=False),
    )
    return kfn(h, src3, dst3)


DA = 2 * DE


def _sc_edge_body(ea_hbm, dst_hbm, out_hbm,
                  dst_v, e0, e1, e2, e3, zbuf, acc,
                  s0, s1, s2, s3):
    cid = lax.axis_index("c")
    sid = lax.axis_index("s")
    wid = sid * NC + cid
    ebufs = [e0, e1, e2, e3]
    sems = [s0, s1, s2, s3]
    EB = 4

    def _zfill(r, _):
        for j in range(DA // 16):
            zbuf[r, pl.ds(j * 16, 16)] = jnp.zeros((16,), f32)
        return _
    lax.fori_loop(0, ZCH, _zfill, None)

    def _ofill(r, _):
        for eb in ebufs:
            eb[r, pl.ds(DE, DE)] = jnp.ones((DE,), f32)
        return _
    lax.fori_loop(0, CH, _ofill, None)
    row0 = sid * RPT
    for k in range(RPT // ZCH):
        pltpu.sync_copy(zbuf, acc.at[pl.ds(row0 + k * ZCH, ZCH)])
    plsc.subcore_barrier()

    pltpu.sync_copy(dst_hbm.at[wid], dst_v)
    ebase = wid * EPW

    for b in range(EB):
        pltpu.async_copy(ea_hbm.at[pl.ds(ebase + b * CH, CH)],
                         ebufs[b].at[:, pl.ds(0, DE)], sems[b])

    def _grp(g, _):
        base = g * EB
        for b in range(EB):
            c = base + b
            pltpu.make_async_copy(ea_hbm.at[pl.ds(ebase + c * CH, CH)],
                                  ebufs[b].at[:, pl.ds(0, DE)], sems[b]).wait()
            pltpu.sync_copy(ebufs[b], acc.at[dst_v.at[c]], add=True)

            @pl.when(c + EB < NCHUNK)
            def _():
                pltpu.async_copy(ea_hbm.at[pl.ds(ebase + (c + EB) * CH, CH)],
                                 ebufs[b].at[:, pl.ds(0, DE)], sems[b])
        return _
    lax.fori_loop(0, NCHUNK // EB, _grp, None)

    plsc.subcore_barrier()
    pltpu.sync_copy(acc.at[pl.ds(row0, RPT)], out_hbm.at[cid, pl.ds(row0, RPT)])


def _sc_edge(edge_attr, dst3):
    mesh = plsc.VectorSubcoreMesh(core_axis_name="c", subcore_axis_name="s")
    kfn = pl.kernel(
        _sc_edge_body,
        out_type=jax.ShapeDtypeStruct((NC, NP, DA), f32),
        mesh=mesh,
        scratch_types=(
            [pltpu.VMEM((NCHUNK, CH), jnp.int32)]
            + [pltpu.VMEM((CH, DA), f32) for _ in range(4)]
            + [pltpu.VMEM((ZCH, DA), f32),
               pltpu.VMEM_SHARED((NP, DA), f32)]
            + [pltpu.SemaphoreType.DMA for _ in range(4)]
        ),
        compiler_params=pltpu.CompilerParams(use_tc_tiling_on_sc=False),
    )
    return kfn(edge_attr, dst3)


bf16 = jnp.bfloat16


def _mm1(a, b):
    return lax.dot_general(a, b, (((1,), (0,)), ((), ())),
                           preferred_element_type=f32,
                           precision=lax.Precision.DEFAULT)


def _mm(a, b):
    a_hi = a.astype(bf16)
    a_lo = (a - a_hi.astype(f32)).astype(bf16)
    b_hi = b.astype(bf16)
    b_lo = (b - b_hi.astype(f32)).astype(bf16)
    return _mm1(a_hi, b_hi) + _mm1(a_lo, b_hi) + _mm1(a_hi, b_lo)


def _mm_a2(a, b_bf):
    a_hi = a.astype(bf16)
    a_lo = (a - a_hi.astype(f32)).astype(bf16)
    return _mm1(a_hi, b_bf) + _mm1(a_lo, b_bf)


def _pool_mm(mask_b, u):
    u_hi = u.astype(bf16)
    u_lo = (u - u_hi.astype(f32)).astype(bf16)
    return _mm1(mask_b, u_hi) + _mm1(mask_b, u_lo)


def _ext_mlp(gmean, W0, b0, W1, b1, W2, b2):
    o = _mm(gmean, W0[...]) + b0[...]
    hh = jnp.maximum(_mm(o, W1[...]) + b1[...], 0.0)
    return _mm(hh, W2[...]) + b2[...]


def _graph_mean(mtb, cnt, u):
    gsum = _pool_mm(mtb, u)
    return gsum / jnp.maximum(cnt, 1.0)


def _tc_pre_body(x_ref, bcol_ref, brow_ref, eadg_ref,
                 Wng0, bng0, Wng1, bng1, Wng2, bng2, Wi, bi,
                 Wm0, bm0, al0, We0, be0,
                 out0_ref, vh_ref, cm_ref, mtb_ref, cnt_ref):
    x = x_ref[...]
    brow = brow_ref[...]
    bcol = bcol_ref[...]
    eadg = eadg_ref[...]
    ea = eadg[:, 0:DE]
    deg1 = eadg[:, DE:DE + 1]

    gids = lax.broadcasted_iota(jnp.int32, (GP, 1), 0)
    mtb = jnp.where(gids == brow, 1.0, 0.0).astype(bf16)
    mtb_ref[...] = mtb
    cnt = jnp.sum(mtb.astype(f32), axis=1, keepdims=True)
    cnt_ref[...] = cnt

    gmean_x = _graph_mean(mtb, cnt, x)
    out0_ref[...] = _ext_mlp(gmean_x, Wng0, bng0, Wng1, bng1, Wng2, bng2)

    gsides = _pool_mm(mtb, eadg)
    gdeg = gsides[:, DE:DE + 1]
    inv_g = 1.0 / jnp.maximum(gdeg / jnp.maximum(cnt, 1.0), 1e-6)

    grow = lax.broadcasted_iota(jnp.int32, (1, GP), 1)
    mnb = jnp.where(bcol == grow, 1.0, 0.0).astype(bf16)
    iv_hi = inv_g.astype(bf16)
    iv_lo = (inv_g - iv_hi.astype(f32)).astype(bf16)
    invb = (_mm1(mnb, jnp.broadcast_to(iv_hi, (GP, DE)))
            + _mm1(mnb, jnp.broadcast_to(iv_lo, (GP, DE))))
    deg16 = jnp.broadcast_to(deg1, (N, DE))

    v = _mm(x, Wi[...]) + bi[...]
    hh = _mm(v, Wm0[...]) + bm0[...]
    cc = (deg1 * (hh * al0[...]) + _mm_a2(ea, We0[...].astype(bf16))
          + deg1 * be0[...])
    meta = jnp.concatenate([ea, deg16, invb, deg16], axis=1)
    vh_ref[...] = jnp.concatenate([v, hh], axis=1)
    cm_ref[...] = jnp.concatenate([cc, meta], axis=1)


def _layer_update(s, cm, vh):
    cc = cm[:, 0:H]
    inv1 = cm[:, H + 2 * DE:H + 2 * DE + 1]
    v = vh[:, 0:H]
    return (s + cc) * inv1 + v


def _tc_mid_body(sp_ref, cm_ref, vh_ref, mtb_ref, cnt_ref,
                 gam, bet, Wm, bm, al, We, be,
                 Wfe0, bfe0, Wfe1, bfe1, Wfe2, bfe2,
                 ext_ref, vh_out_ref, cm_out_ref):
    cm = cm_ref[...]
    u = _layer_update(sp_ref[...], cm, vh_ref[...])
    gmean = _graph_mean(mtb_ref[...], cnt_ref[...], u)
    ext_ref[...] = _ext_mlp(gmean, Wfe0, bfe0, Wfe1, bfe1, Wfe2, bfe2)

    m = jnp.mean(u, axis=0, keepdims=True)
    var = jnp.mean((u - m) ** 2, axis=0, keepdims=True)
    vn = (u - m) / jnp.sqrt(var + EPS) * gam[...] + bet[...]

    meta = cm[:, H:]
    ea = meta[:, 0:DE]
    deg1 = meta[:, DE:DE + 1]
    hh = _mm(vn, Wm[...]) + bm[...]
    cc = (deg1 * (hh * al[...]) + _mm_a2(ea, We[...].astype(bf16))
          + deg1 * be[...])
    vh_out_ref[...] = jnp.concatenate([vn, hh], axis=1)
    cm_out_ref[...] = jnp.concatenate([cc, meta], axis=1)


def _tc_final_body(sp_ref, cm_ref, vh_ref, mtb_ref, cnt_ref, out0_ref,
                   e0, e1, e2,
                   Wfe0, bfe0, Wfe1, bfe1, Wfe2, bfe2, Wa, ba, Wl, bl,
                   res_ref):
    u = _layer_update(sp_ref[...], cm_ref[...], vh_ref[...])
    gmean = _graph_mean(mtb_ref[...], cnt_ref[...], u)
    e3 = _ext_mlp(gmean, Wfe0, bfe0, Wfe1, bfe1, Wfe2, bfe2)
    out = out0_ref[...] + (e0[...] + e1[...] + e2[...] + e3) * (1.0 / NL)
    o2 = jnp.maximum(_mm(out, Wa[...]) + ba[...], 0.0) + out
    res_ref[...] = _mm(o2, Wl[...]) + bl[...]


def _sds(shape):
    return jax.ShapeDtypeStruct(shape, f32)


def kernel(x, edge_attr, edge_index, batch,
           Wng0, bng0, Wng1, bng1, Wng2, bng2, Wi, bi,
           Wm, bm, alpha, We, be, gamma, beta,
           Wfe0, bfe0, Wfe1, bfe1, Wfe2, bfe2, Wa, ba, Wl, bl):
    src3 = edge_index[0].reshape(NW, NCHUNK, CH)
    dst3 = edge_index[1].reshape(NW, NCHUNK, CH)
    bcol = batch.reshape(N, 1)
    brow = batch.reshape(1, N)

    def r2(b):
        return b.reshape(1, -1)

    eadg3 = _sc_edge(edge_attr, dst3)
    eadg = eadg3[0, :N] + eadg3[1, :N]

    out0, vh, cm, mtb, cnt = pl.pallas_call(
        _tc_pre_body,
        out_shape=(_sds((GP, H)), _sds((N, 2 * H)), _sds((N, 2 * H)),
                   jax.ShapeDtypeStruct((GP, N), jnp.bfloat16), _sds((GP, 1))),
    )(x, bcol, brow, eadg,
      Wng0, r2(bng0), Wng1, r2(bng1), Wng2, r2(bng2), Wi, r2(bi),
      Wm[0], r2(bm[0]), r2(alpha[0]), We[0], r2(be[0]))

    exts = []
    for i in range(NL - 1):
        Sp = _sc_gather(vh[:, H:], src3, dst3)
        ss = Sp[0, :N] + Sp[1, :N]
        ext_i, vh, cm = pl.pallas_call(
            _tc_mid_body,
            out_shape=(_sds((GP, H)), _sds((N, 2 * H)), _sds((N, 2 * H))),
        )(ss, cm, vh, mtb, cnt,
          r2(gamma[i]), r2(beta[i]),
          Wm[i + 1], r2(bm[i + 1]), r2(alpha[i + 1]), We[i + 1], r2(be[i + 1]),
          Wfe0, r2(bfe0), Wfe1, r2(bfe1), Wfe2, r2(bfe2))
        exts.append(ext_i)

    Sp = _sc_gather(vh[:, H:], src3, dst3)
    ss = Sp[0, :N] + Sp[1, :N]
    res = pl.pallas_call(
        _tc_final_body,
        out_shape=_sds((GP, 1)),
    )(ss, cm, vh, mtb, cnt, out0, exts[0], exts[1], exts[2],
      Wfe0, r2(bfe0), Wfe1, r2(bfe1), Wfe2, r2(bfe2),
      Wa, r2(ba), Wl, r2(bl))

    return res[:G]


# speedup vs baseline: 15.7498x; 1.0206x over previous
import functools

import jax
import jax.numpy as jnp
from jax import lax
from jax.experimental import pallas as pl
from jax.experimental.pallas import tpu as pltpu
from jax.experimental.pallas import tpu_sc as plsc

N = 10000
E = 320000
G = 100
GP = 128
DIN = 128
DE = 16
H = 64
NL = 4
EPS = 1e-5

NC = 2
NS = 16
NW = NC * NS
EPW = E // NW
CH = 125
NCHUNK = EPW // CH
NB = 4
NP = 10240
RPT = NP // NS
ZCH = 128

f32 = jnp.float32


def _sc_gather_body(h_hbm, src_hbm, dst_hbm, out_hbm,
                    src_v, dst_v, r0, r1, r2, r3, zbuf, acc,
                    s0, s1, s2, s3):
    cid = lax.axis_index("c")
    sid = lax.axis_index("s")
    wid = sid * NC + cid
    rows = [r0, r1, r2, r3]
    sems = [s0, s1, s2, s3]

    def _zrow(r, _):
        for j in range(H // 16):
            zbuf[r, pl.ds(j * 16, 16)] = jnp.zeros((16,), f32)
        return _
    lax.fori_loop(0, ZCH, _zrow, None)
    row0 = sid * RPT
    for k in range(RPT // ZCH):
        pltpu.sync_copy(zbuf, acc.at[pl.ds(row0 + k * ZCH, ZCH)])
    plsc.subcore_barrier()

    pltpu.sync_copy(src_hbm.at[wid], src_v)
    pltpu.sync_copy(dst_hbm.at[wid], dst_v)

    for b in range(NB):
        pltpu.async_copy(h_hbm.at[src_v.at[b]], rows[b], sems[b])

    def _grp(g, _):
        base = g * NB
        for b in range(NB):
            c = base + b
            pltpu.make_async_copy(h_hbm.at[src_v.at[c]], rows[b], sems[b]).wait()
            pltpu.sync_copy(rows[b], acc.at[dst_v.at[c]], add=True)

            @pl.when(c + NB < NCHUNK)
            def _():
                pltpu.async_copy(h_hbm.at[src_v.at[c + NB]], rows[b], sems[b])
        return _
    lax.fori_loop(0, NCHUNK // NB, _grp, None)

    plsc.subcore_barrier()
    pltpu.sync_copy(acc.at[pl.ds(row0, RPT)], out_hbm.at[cid, pl.ds(row0, RPT)])


def _sc_gather(h, src3, dst3):
    mesh = plsc.VectorSubcoreMesh(core_axis_name="c", subcore_axis_name="s")
    kfn = pl.kernel(
        _sc_gather_body,
        out_type=jax.ShapeDtypeStruct((NC, NP, H), f32),
        mesh=mesh,
        scratch_types=(
            [pltpu.VMEM((NCHUNK, CH), jnp.int32),
             pltpu.VMEM((NCHUNK, CH), jnp.int32)]
            + [pltpu.VMEM((CH, H), f32) for _ in range(NB)]
            + [pltpu.VMEM((ZCH, H), f32),
               pltpu.VMEM_SHARED((NP, H), f32)]
            + [pltpu.SemaphoreType.DMA for _ in range(NB)]
        ),
        compiler_params=pltpu.CompilerParams(use_tc_tiling_on_sc=False),
    )
    return kfn(h, src3, dst3)


DA = 2 * DE


ECH = 128
NEC = 78
ERPC = ECH // 8
NECP = 79


def _sc_edge_body(ea_hbm, dst_hbm, out_hbm,
                  dst_v, e0, e1, e2, e3, tl, sbuf, zbuf, acc,
                  s0, s1, s2, s3, st):
    cid = lax.axis_index("c")
    sid = lax.axis_index("s")
    wid = sid * NC + cid
    ebufs = [e0, e1, e2, e3]
    sems = [s0, s1, s2, s3]
    EB = 4

    def _zfill(r, _):
        for j in range(DA // 16):
            zbuf[r, pl.ds(j * 16, 16)] = jnp.zeros((16,), f32)
        return _
    lax.fori_loop(0, ZCH, _zfill, None)

    def _ofill(r, _):
        sbuf[r, pl.ds(DE, DE)] = jnp.ones((DE,), f32)
        return _
    lax.fori_loop(0, ECH, _ofill, None)
    row0 = sid * RPT
    for k in range(RPT // ZCH):
        pltpu.sync_copy(zbuf, acc.at[pl.ds(row0 + k * ZCH, ZCH)])
    plsc.subcore_barrier()

    pltpu.sync_copy(dst_hbm.at[wid], dst_v)
    rbase = wid * (EPW // 8)

    for b in range(EB):
        pltpu.async_copy(ea_hbm.at[pl.ds(rbase + b * ERPC, ERPC)],
                         ebufs[b], sems[b])

    def _repack(ld, nrow):
        def _row(r, _):
            for j in range(8):
                sbuf[r * 8 + j, pl.ds(0, DE)] = ld[r, pl.ds(j * DE, DE)]
            return _
        lax.fori_loop(0, nrow, _row, None)

    def _grp(g, _):
        base = g * EB
        for b in range(EB):
            c = base + b
            pltpu.make_async_copy(ea_hbm.at[pl.ds(rbase + c * ERPC, ERPC)],
                                  ebufs[b], sems[b]).wait()
            _repack(ebufs[b], ERPC)
            pltpu.sync_copy(sbuf, acc.at[dst_v.at[c]], add=True)

            @pl.when(c + EB < NEC)
            def _():
                pltpu.async_copy(ea_hbm.at[pl.ds(rbase + (c + EB) * ERPC, ERPC)],
                                 ebufs[b], sems[b])
        return _
    lax.fori_loop(0, NEC // EB, _grp, None)
    for c in range(NEC - NEC % EB, NEC):
        b = c % EB
        pltpu.make_async_copy(ea_hbm.at[pl.ds(rbase + c * ERPC, ERPC)],
                              ebufs[b], sems[b]).wait()
        _repack(ebufs[b], ERPC)
        pltpu.sync_copy(sbuf, acc.at[dst_v.at[c]], add=True)

    pltpu.async_copy(ea_hbm.at[pl.ds(rbase + NEC * ERPC, 2)], tl, st)
    pltpu.make_async_copy(ea_hbm.at[pl.ds(rbase + NEC * ERPC, 2)], tl, st).wait()
    _repack(tl, 2)
    pltpu.sync_copy(sbuf, acc.at[dst_v.at[NEC]], add=True)

    plsc.subcore_barrier()
    pltpu.sync_copy(acc.at[pl.ds(row0, RPT)], out_hbm.at[cid, pl.ds(row0, RPT)])


def _sc_edge(ea2, dstp):
    mesh = plsc.VectorSubcoreMesh(core_axis_name="c", subcore_axis_name="s")
    kfn = pl.kernel(
        _sc_edge_body,
        out_type=jax.ShapeDtypeStruct((NC, NP, DA), f32),
        mesh=mesh,
        scratch_types=(
            [pltpu.VMEM((NECP, ECH), jnp.int32)]
            + [pltpu.VMEM((ERPC, 8 * DE), f32) for _ in range(4)]
            + [pltpu.VMEM((2, 8 * DE), f32),
               pltpu.VMEM((ECH, DA), f32),
               pltpu.VMEM((ZCH, DA), f32),
               pltpu.VMEM_SHARED((NP, DA), f32)]
            + [pltpu.SemaphoreType.DMA for _ in range(5)]
        ),
        compiler_params=pltpu.CompilerParams(use_tc_tiling_on_sc=False),
    )
    return kfn(ea2, dstp)


bf16 = jnp.bfloat16


def _mm1(a, b):
    return lax.dot_general(a, b, (((1,), (0,)), ((), ())),
                           preferred_element_type=f32,
                           precision=lax.Precision.DEFAULT)


def _mm(a, b):
    a_hi = a.astype(bf16)
    a_lo = (a - a_hi.astype(f32)).astype(bf16)
    b_hi = b.astype(bf16)
    b_lo = (b - b_hi.astype(f32)).astype(bf16)
    return _mm1(a_hi, b_hi) + _mm1(a_lo, b_hi) + _mm1(a_hi, b_lo)


def _mm_a2(a, b_bf):
    a_hi = a.astype(bf16)
    a_lo = (a - a_hi.astype(f32)).astype(bf16)
    return _mm1(a_hi, b_bf) + _mm1(a_lo, b_bf)


def _pool_mm(mask_b, u):
    u_hi = u.astype(bf16)
    u_lo = (u - u_hi.astype(f32)).astype(bf16)
    return _mm1(mask_b, u_hi) + _mm1(mask_b, u_lo)


def _ext_mlp(gmean, W0, b0, W1, b1, W2, b2):
    o = _mm(gmean, W0[...]) + b0[...]
    hh = jnp.maximum(_mm(o, W1[...]) + b1[...], 0.0)
    return _mm(hh, W2[...]) + b2[...]


def _graph_mean(mtb, cnt, u):
    gsum = _pool_mm(mtb, u)
    return gsum / jnp.maximum(cnt, 1.0)


def _tc_pre_body(x_ref, bcol_ref, brow_ref, eadg_ref,
                 Wng0, bng0, Wng1, bng1, Wng2, bng2, Wi, bi,
                 Wm0, bm0, al0, We0, be0,
                 out0_ref, vh_ref, cm_ref, mtb_ref, cnt_ref):
    x = x_ref[...]
    brow = brow_ref[...]
    bcol = bcol_ref[...]
    eadg = eadg_ref[...]
    ea = eadg[:, 0:DE]
    deg1 = eadg[:, DE:DE + 1]

    gids = lax.broadcasted_iota(jnp.int32, (GP, 1), 0)
    mtb = jnp.where(gids == brow, 1.0, 0.0).astype(bf16)
    mtb_ref[...] = mtb
    cnt = jnp.sum(mtb.astype(f32), axis=1, keepdims=True)
    cnt_ref[...] = cnt

    gmean_x = _graph_mean(mtb, cnt, x)
    out0_ref[...] = _ext_mlp(gmean_x, Wng0, bng0, Wng1, bng1, Wng2, bng2)

    gsides = _pool_mm(mtb, eadg)
    gdeg = gsides[:, DE:DE + 1]
    inv_g = 1.0 / jnp.maximum(gdeg / jnp.maximum(cnt, 1.0), 1e-6)

    grow = lax.broadcasted_iota(jnp.int32, (1, GP), 1)
    mnb = jnp.where(bcol == grow, 1.0, 0.0).astype(bf16)
    iv_hi = inv_g.astype(bf16)
    iv_lo = (inv_g - iv_hi.astype(f32)).astype(bf16)
    invb = (_mm1(mnb, jnp.broadcast_to(iv_hi, (GP, DE)))
            + _mm1(mnb, jnp.broadcast_to(iv_lo, (GP, DE))))
    deg16 = jnp.broadcast_to(deg1, (N, DE))

    v = _mm(x, Wi[...]) + bi[...]
    hh = _mm(v, Wm0[...]) + bm0[...]
    cc = (deg1 * (hh * al0[...]) + _mm_a2(ea, We0[...].astype(bf16))
          + deg1 * be0[...])
    meta = jnp.concatenate([ea, deg16, invb, deg16], axis=1)
    vh_ref[...] = jnp.concatenate([v, hh], axis=1)
    cm_ref[...] = jnp.concatenate([cc, meta], axis=1)


def _layer_update(s, cm, vh):
    cc = cm[:, 0:H]
    inv1 = cm[:, H + 2 * DE:H + 2 * DE + 1]
    v = vh[:, 0:H]
    return (s + cc) * inv1 + v


def _tc_mid_body(sp_ref, cm_ref, vh_ref, mtb_ref, cnt_ref,
                 gam, bet, Wm, bm, al, We, be,
                 Wfe0, bfe0, Wfe1, bfe1, Wfe2, bfe2,
                 ext_ref, vh_out_ref, cm_out_ref):
    cm = cm_ref[...]
    u = _layer_update(sp_ref[...], cm, vh_ref[...])
    gmean = _graph_mean(mtb_ref[...], cnt_ref[...], u)
    ext_ref[...] = _ext_mlp(gmean, Wfe0, bfe0, Wfe1, bfe1, Wfe2, bfe2)

    m = jnp.mean(u, axis=0, keepdims=True)
    var = jnp.mean((u - m) ** 2, axis=0, keepdims=True)
    vn = (u - m) / jnp.sqrt(var + EPS) * gam[...] + bet[...]

    meta = cm[:, H:]
    ea = meta[:, 0:DE]
    deg1 = meta[:, DE:DE + 1]
    hh = _mm(vn, Wm[...]) + bm[...]
    cc = (deg1 * (hh * al[...]) + _mm_a2(ea, We[...].astype(bf16))
          + deg1 * be[...])
    vh_out_ref[...] = jnp.concatenate([vn, hh], axis=1)
    cm_out_ref[...] = jnp.concatenate([cc, meta], axis=1)


def _tc_final_body(sp_ref, cm_ref, vh_ref, mtb_ref, cnt_ref, out0_ref,
                   e0, e1, e2,
                   Wfe0, bfe0, Wfe1, bfe1, Wfe2, bfe2, Wa, ba, Wl, bl,
                   res_ref):
    u = _layer_update(sp_ref[...], cm_ref[...], vh_ref[...])
    gmean = _graph_mean(mtb_ref[...], cnt_ref[...], u)
    e3 = _ext_mlp(gmean, Wfe0, bfe0, Wfe1, bfe1, Wfe2, bfe2)
    out = out0_ref[...] + (e0[...] + e1[...] + e2[...] + e3) * (1.0 / NL)
    o2 = jnp.maximum(_mm(out, Wa[...]) + ba[...], 0.0) + out
    res_ref[...] = _mm(o2, Wl[...]) + bl[...]


def _sds(shape):
    return jax.ShapeDtypeStruct(shape, f32)


def kernel(x, edge_attr, edge_index, batch,
           Wng0, bng0, Wng1, bng1, Wng2, bng2, Wi, bi,
           Wm, bm, alpha, We, be, gamma, beta,
           Wfe0, bfe0, Wfe1, bfe1, Wfe2, bfe2, Wa, ba, Wl, bl):
    src3 = edge_index[0].reshape(NW, NCHUNK, CH)
    dst3 = edge_index[1].reshape(NW, NCHUNK, CH)
    ea2 = edge_attr.reshape(E // 8, 8 * DE)
    dstp = jnp.pad(edge_index[1].reshape(NW, EPW), ((0, 0), (0, 112)),
                   constant_values=N).reshape(NW, NECP, ECH)
    bcol = batch.reshape(N, 1)
    brow = batch.reshape(1, N)

    def r2(b):
        return b.reshape(1, -1)

    eadg3 = _sc_edge(ea2, dstp)
    eadg = eadg3[0, :N] + eadg3[1, :N]

    out0, vh, cm, mtb, cnt = pl.pallas_call(
        _tc_pre_body,
        out_shape=(_sds((GP, H)), _sds((N, 2 * H)), _sds((N, 2 * H)),
                   jax.ShapeDtypeStruct((GP, N), jnp.bfloat16), _sds((GP, 1))),
    )(x, bcol, brow, eadg,
      Wng0, r2(bng0), Wng1, r2(bng1), Wng2, r2(bng2), Wi, r2(bi),
      Wm[0], r2(bm[0]), r2(alpha[0]), We[0], r2(be[0]))

    exts = []
    for i in range(NL - 1):
        Sp = _sc_gather(vh[:, H:], src3, dst3)
        ss = Sp[0, :N] + Sp[1, :N]
        ext_i, vh, cm = pl.pallas_call(
            _tc_mid_body,
            out_shape=(_sds((GP, H)), _sds((N, 2 * H)), _sds((N, 2 * H))),
        )(ss, cm, vh, mtb, cnt,
          r2(gamma[i]), r2(beta[i]),
          Wm[i + 1], r2(bm[i + 1]), r2(alpha[i + 1]), We[i + 1], r2(be[i + 1]),
          Wfe0, r2(bfe0), Wfe1, r2(bfe1), Wfe2, r2(bfe2))
        exts.append(ext_i)

    Sp = _sc_gather(vh[:, H:], src3, dst3)
    ss = Sp[0, :N] + Sp[1, :N]
    res = pl.pallas_call(
        _tc_final_body,
        out_shape=_sds((GP, 1)),
    )(ss, cm, vh, mtb, cnt, out0, exts[0], exts[1], exts[2],
      Wfe0, r2(bfe0), Wfe1, r2(bfe1), Wfe2, r2(bfe2),
      Wa, r2(ba), Wl, r2(bl))

    return res[:G]
